# baseline (speedup 1.0000x reference)
import functools

import jax
import jax.numpy as jnp
from jax.experimental import pallas as pl
from jax.experimental.pallas import tpu as pltpu

_CUTOFF = 10


def _exact_threshold(data, rem_ref, k):
    t = data.shape[0]
    rem_ref[...] = data

    def body(_, carry):
        thresh, needed = carry
        rem = rem_ref[...]
        m = jnp.max(rem, axis=1, keepdims=True)
        eq = rem == m
        cnt = jnp.sum(eq.astype(jnp.float32), axis=1, keepdims=True)
        hit = jnp.logical_and(needed > 0.0, cnt >= needed)
        thresh = jnp.where(hit, m, thresh)
        needed = needed - cnt
        rem_ref[...] = jnp.where(eq, -2.0, rem)
        return thresh, needed

    thresh0 = jnp.full((t, 1), -jnp.inf, jnp.float32)
    needed0 = jnp.full((t, 1), float(k), jnp.float32)
    thresh, _ = jax.lax.fori_loop(0, k, body, (thresh0, needed0))
    return thresh


def _lane_pair_reduce(v, p):
    w = v.shape[1]
    while w > 1:
        h = w // 2
        av, bv = v[:, :h], v[:, h:w]
        ap, bp = p[:, :h], p[:, h:w]
        take = av >= bv
        v = jnp.where(take, av, bv)
        p = jnp.where(take, ap, bp)
        w = h
    return v, p


def _top4_insert(m1, m2, m3, m4, x):
    n1 = jnp.maximum(m1, x)
    l1 = jnp.minimum(m1, x)
    n2 = jnp.maximum(m2, l1)
    l2 = jnp.minimum(m2, l1)
    n3 = jnp.maximum(m3, l2)
    l3 = jnp.minimum(m3, l2)
    n4 = jnp.maximum(m4, l3)
    return n1, n2, n3, n4


def _fused_body(dist_ref, rep_ref, wlt_ref, b_ref, g_ref, beta_ref, wemb_ref,
                out_ref, rem_ref, thr_ref, *, k, v):
    ts, bb, _ = dist_ref.shape
    t = ts * bb
    dist = dist_ref[...].reshape(t, v)

    neg = jnp.full((t, 128), -1.0, jnp.float32)
    m1, m2, m3, m4 = neg, neg, neg, neg
    nfull = v // 128
    for q in range(nfull):
        x = dist[:, q * 128:(q + 1) * 128]
        m1, m2, m3, m4 = _top4_insert(m1, m2, m3, m4, x)
    tail = v - nfull * 128
    if tail:
        lane = jax.lax.broadcasted_iota(jnp.int32, (t, 128), 1)
        x = jnp.where(lane >= 128 - tail, dist[:, v - 128:v], -1.0)
        m1, m2, m3, m4 = _top4_insert(m1, m2, m3, m4, x)

    lif = jax.lax.broadcasted_iota(jnp.int32, (t, 128), 1).astype(jnp.float32)
    pp = m1
    depth = jnp.zeros((t, 128), jnp.float32)
    thr_fast = None
    for _ in range(k):
        mv, mp = _lane_pair_reduce(pp, lif)
        thr_fast = mv
        hit = lif == mp
        depth = depth + jnp.where(hit, 1.0, 0.0)
        nxt = jnp.where(depth >= 3.5, -2.0,
                        jnp.where(depth >= 2.5, m4,
                                  jnp.where(depth >= 1.5, m3, m2)))
        pp = jnp.where(hit, nxt, pp)

    cgt = jnp.sum((dist > thr_fast).astype(jnp.float32), axis=1,
                  keepdims=True)
    ok = jnp.max(cgt) <= float(k - 1)

    thr_ref[...] = thr_fast

    @pl.when(jnp.logical_not(ok))
    def _slow():
        thr_ref[...] = _exact_threshold(dist, rem_ref, k)

    thresh = thr_ref[...]

    masked = jnp.where(dist > thresh, dist, 0.0).astype(jnp.bfloat16)
    soft = jnp.dot(masked, wemb_ref[...], preferred_element_type=jnp.float32)

    d = rep_ref.shape[-1]
    h = jnp.dot(rep_ref[...].reshape(t, d), wlt_ref[...],
                preferred_element_type=jnp.float32) + b_ref[...]
    mu = jnp.mean(h, axis=1, keepdims=True)
    var = jnp.mean((h - mu) ** 2, axis=1, keepdims=True)
    ln = (h - mu) * jax.lax.rsqrt(var + 1e-5) * g_ref[...] + beta_ref[...]
    out_ref[...] = (jnp.maximum(ln, 0.0) + soft).reshape(ts, bb, d)


def kernel(representation, distribution, W_lin, b_lin, gamma, beta, W_embed):
    s, b, d = representation.shape
    v = distribution.shape[-1]
    n = s * b
    k = min(_CUTOFF, v - 1) + 1

    wemb16 = W_embed.astype(jnp.bfloat16)
    wlt = W_lin.T
    b2 = b_lin.reshape(1, d)
    g2 = gamma.reshape(1, d)
    be2 = beta.reshape(1, d)

    ts = min(max(64 // b, 1), s)
    t = ts * b
    assert s % ts == 0 and v >= 256
    grid = (s // ts,)

    out3d = pl.pallas_call(
        functools.partial(_fused_body, k=k, v=v),
        grid=grid,
        in_specs=[
            pl.BlockSpec((ts, b, v), lambda i: (i, 0, 0)),
            pl.BlockSpec((ts, b, d), lambda i: (i, 0, 0)),
            pl.BlockSpec((d, d), lambda i: (0, 0)),
            pl.BlockSpec((1, d), lambda i: (0, 0)),
            pl.BlockSpec((1, d), lambda i: (0, 0)),
            pl.BlockSpec((1, d), lambda i: (0, 0)),
            pl.BlockSpec((v, d), lambda i: (0, 0)),
        ],
        out_specs=pl.BlockSpec((ts, b, d), lambda i: (i, 0, 0)),
        out_shape=jax.ShapeDtypeStruct((s, b, d), jnp.float32),
        scratch_shapes=[
            pltpu.VMEM((t, v), jnp.float32),
            pltpu.VMEM((t, 1), jnp.float32),
        ],
    )(distribution, representation, wlt, b2, g2, be2, wemb16)

    return out3d


# speedup vs baseline: 1.4032x; 1.4032x over previous
import functools

import jax
import jax.numpy as jnp
from jax.experimental import pallas as pl
from jax.experimental.pallas import tpu as pltpu

_CUTOFF = 10


def _exact_threshold(data, rem_ref, k):
    t = data.shape[0]
    rem_ref[...] = data

    def body(_, carry):
        thresh, needed = carry
        rem = rem_ref[...]
        m = jnp.max(rem, axis=1, keepdims=True)
        eq = rem == m
        cnt = jnp.sum(eq.astype(jnp.float32), axis=1, keepdims=True)
        hit = jnp.logical_and(needed > 0.0, cnt >= needed)
        thresh = jnp.where(hit, m, thresh)
        needed = needed - cnt
        rem_ref[...] = jnp.where(eq, -2.0, rem)
        return thresh, needed

    thresh0 = jnp.full((t, 1), -jnp.inf, jnp.float32)
    needed0 = jnp.full((t, 1), float(k), jnp.float32)
    thresh, _ = jax.lax.fori_loop(0, k, body, (thresh0, needed0))
    return thresh


def _top4_insert(m1, m2, m3, m4, x):
    n1 = jnp.maximum(m1, x)
    l1 = jnp.minimum(m1, x)
    n2 = jnp.maximum(m2, l1)
    l2 = jnp.minimum(m2, l1)
    n3 = jnp.maximum(m3, l2)
    l3 = jnp.minimum(m3, l2)
    n4 = jnp.maximum(m4, l3)
    return n1, n2, n3, n4


def _fused_body(dist_ref, rep_ref, wlt_ref, b_ref, g_ref, beta_ref, wemb_ref,
                out_ref, cand_ref, rem_ref, thr_ref, *, k, v):
    ts, bb, _ = dist_ref.shape
    t = ts * bb
    dist = dist_ref[...].reshape(t, v)

    neg = jnp.full((t, 128), -1.0, jnp.float32)
    m1, m2, m3, m4 = neg, neg, neg, neg
    nfull = v // 128
    for q in range(nfull):
        x = dist[:, q * 128:(q + 1) * 128]
        m1, m2, m3, m4 = _top4_insert(m1, m2, m3, m4, x)
    tail = v - nfull * 128
    if tail:
        lane = jax.lax.broadcasted_iota(jnp.int32, (t, 128), 1)
        x = jnp.where(lane >= 128 - tail, dist[:, v - 128:v], -1.0)
        m1, m2, m3, m4 = _top4_insert(m1, m2, m3, m4, x)

    cand = jnp.concatenate([m1, m2, m3, m4], axis=1)
    thr_fast = _exact_threshold(cand, cand_ref, k)

    cgt = jnp.sum((dist > thr_fast).astype(jnp.float32), axis=1,
                  keepdims=True)
    ok = jnp.max(cgt) <= float(k - 1)

    thr_ref[...] = thr_fast

    @pl.when(jnp.logical_not(ok))
    def _slow():
        thr_ref[...] = _exact_threshold(dist, rem_ref, k)

    thresh = thr_ref[...]

    masked = jnp.where(dist > thresh, dist, 0.0).astype(jnp.bfloat16)
    soft = jnp.dot(masked, wemb_ref[...], preferred_element_type=jnp.float32)

    d = rep_ref.shape[-1]
    h = jnp.dot(rep_ref[...].reshape(t, d), wlt_ref[...],
                preferred_element_type=jnp.float32) + b_ref[...]
    mu = jnp.mean(h, axis=1, keepdims=True)
    var = jnp.mean((h - mu) ** 2, axis=1, keepdims=True)
    ln = (h - mu) * jax.lax.rsqrt(var + 1e-5) * g_ref[...] + beta_ref[...]
    out_ref[...] = (jnp.maximum(ln, 0.0) + soft).reshape(ts, bb, d)


def kernel(representation, distribution, W_lin, b_lin, gamma, beta, W_embed):
    s, b, d = representation.shape
    v = distribution.shape[-1]
    n = s * b
    k = min(_CUTOFF, v - 1) + 1

    wemb16 = W_embed.astype(jnp.bfloat16)
    wlt = W_lin.T
    b2 = b_lin.reshape(1, d)
    g2 = gamma.reshape(1, d)
    be2 = beta.reshape(1, d)

    ts = min(max(64 // b, 1), s)
    t = ts * b
    assert s % ts == 0 and v >= 256
    grid = (s // ts,)

    out3d = pl.pallas_call(
        functools.partial(_fused_body, k=k, v=v),
        grid=grid,
        in_specs=[
            pl.BlockSpec((ts, b, v), lambda i: (i, 0, 0)),
            pl.BlockSpec((ts, b, d), lambda i: (i, 0, 0)),
            pl.BlockSpec((d, d), lambda i: (0, 0)),
            pl.BlockSpec((1, d), lambda i: (0, 0)),
            pl.BlockSpec((1, d), lambda i: (0, 0)),
            pl.BlockSpec((1, d), lambda i: (0, 0)),
            pl.BlockSpec((v, d), lambda i: (0, 0)),
        ],
        out_specs=pl.BlockSpec((ts, b, d), lambda i: (i, 0, 0)),
        out_shape=jax.ShapeDtypeStruct((s, b, d), jnp.float32),
        scratch_shapes=[
            pltpu.VMEM((t, 512), jnp.float32),
            pltpu.VMEM((t, v), jnp.float32),
            pltpu.VMEM((t, 1), jnp.float32),
        ],
    )(distribution, representation, wlt, b2, g2, be2, wemb16)

    return out3d


# speedup vs baseline: 1.9298x; 1.3753x over previous
import functools

import jax
import jax.numpy as jnp
from jax.experimental import pallas as pl
from jax.experimental.pallas import tpu as pltpu

_CUTOFF = 10


def _exact_threshold(data, rem_ref, k):
    t = data.shape[0]
    rem_ref[...] = data

    def body(_, carry):
        thresh, needed = carry
        rem = rem_ref[...]
        m = jnp.max(rem, axis=1, keepdims=True)
        eq = rem == m
        cnt = jnp.sum(eq.astype(jnp.float32), axis=1, keepdims=True)
        hit = jnp.logical_and(needed > 0.0, cnt >= needed)
        thresh = jnp.where(hit, m, thresh)
        needed = needed - cnt
        rem_ref[...] = jnp.where(eq, -2.0, rem)
        return thresh, needed

    thresh0 = jnp.full((t, 1), -jnp.inf, jnp.float32)
    needed0 = jnp.full((t, 1), float(k), jnp.float32)
    thresh, _ = jax.lax.fori_loop(0, k, body, (thresh0, needed0))
    return thresh


def _top4_insert(m1, m2, m3, m4, x):
    n1 = jnp.maximum(m1, x)
    l1 = jnp.minimum(m1, x)
    n2 = jnp.maximum(m2, l1)
    l2 = jnp.minimum(m2, l1)
    n3 = jnp.maximum(m3, l2)
    l3 = jnp.minimum(m3, l2)
    n4 = jnp.maximum(m4, l3)
    return n1, n2, n3, n4


def _fused_body(dist_ref, rep_ref, wlt_ref, b_ref, g_ref, beta_ref, wemb_ref,
                out_ref, cand_ref, rem_ref, thr_ref, *, k, v):
    ts, bb, _ = dist_ref.shape
    t = ts * bb
    dist = dist_ref[...].reshape(t, v)

    neg = jnp.full((t, 128), -1.0, jnp.float32)
    m1, m2, m3, m4 = neg, neg, neg, neg
    nfull = v // 128
    for q in range(nfull):
        x = dist[:, q * 128:(q + 1) * 128]
        m1, m2, m3, m4 = _top4_insert(m1, m2, m3, m4, x)
    tail = v - nfull * 128
    if tail:
        lane = jax.lax.broadcasted_iota(jnp.int32, (t, 128), 1)
        x = jnp.where(lane >= 128 - tail, dist[:, v - 128:v], -1.0)
        m1, m2, m3, m4 = _top4_insert(m1, m2, m3, m4, x)

    cand = jnp.concatenate([m1, m2, m3, m4], axis=1)
    thr_fast = _exact_threshold(cand, cand_ref, k)

    cgt = jnp.sum((dist > thr_fast).astype(jnp.float32), axis=1,
                  keepdims=True)
    ok = jnp.max(cgt) <= float(k - 1)

    thr_ref[...] = thr_fast

    @pl.when(jnp.logical_not(ok))
    def _slow():
        thr_ref[...] = _exact_threshold(dist, rem_ref, k)

    thresh = thr_ref[...]

    masked = jnp.where(dist > thresh, dist, 0.0).astype(jnp.bfloat16)
    soft = jnp.dot(masked, wemb_ref[...], preferred_element_type=jnp.float32)

    d = rep_ref.shape[-1]
    h = jnp.dot(rep_ref[...].reshape(t, d), wlt_ref[...],
                preferred_element_type=jnp.float32) + b_ref[...]
    mu = jnp.mean(h, axis=1, keepdims=True)
    var = jnp.mean((h - mu) ** 2, axis=1, keepdims=True)
    ln = (h - mu) * jax.lax.rsqrt(var + 1e-5) * g_ref[...] + beta_ref[...]
    out_ref[...] = (jnp.maximum(ln, 0.0) + soft).reshape(ts, bb, d)


def kernel(representation, distribution, W_lin, b_lin, gamma, beta, W_embed):
    s, b, d = representation.shape
    v = distribution.shape[-1]
    n = s * b
    k = min(_CUTOFF, v - 1) + 1

    wemb16 = W_embed.astype(jnp.bfloat16)
    wlt = W_lin.T
    b2 = b_lin.reshape(1, d)
    g2 = gamma.reshape(1, d)
    be2 = beta.reshape(1, d)

    ts = min(max(128 // b, 1), s)
    t = ts * b
    assert s % ts == 0 and v >= 256
    grid = (s // ts,)

    out3d = pl.pallas_call(
        functools.partial(_fused_body, k=k, v=v),
        grid=grid,
        in_specs=[
            pl.BlockSpec((ts, b, v), lambda i: (i, 0, 0)),
            pl.BlockSpec((ts, b, d), lambda i: (i, 0, 0)),
            pl.BlockSpec((d, d), lambda i: (0, 0)),
            pl.BlockSpec((1, d), lambda i: (0, 0)),
            pl.BlockSpec((1, d), lambda i: (0, 0)),
            pl.BlockSpec((1, d), lambda i: (0, 0)),
            pl.BlockSpec((v, d), lambda i: (0, 0)),
        ],
        out_specs=pl.BlockSpec((ts, b, d), lambda i: (i, 0, 0)),
        out_shape=jax.ShapeDtypeStruct((s, b, d), jnp.float32),
        scratch_shapes=[
            pltpu.VMEM((t, 512), jnp.float32),
            pltpu.VMEM((t, v), jnp.float32),
            pltpu.VMEM((t, 1), jnp.float32),
        ],
    )(distribution, representation, wlt, b2, g2, be2, wemb16)

    return out3d
